# baseline (speedup 1.0000x reference)
import functools

import jax
import jax.numpy as jnp
from jax import lax
from jax.experimental import pallas as pl
from jax.experimental.pallas import tpu as pltpu
from jax.experimental.pallas import tpu_sc as plsc

N = 10000
E = 320000
IN_DIM = 128
GEO = 64
HID = 32
DIMS = [32, 64, 128, 256, 128, 64, 32]

NT = 16
EPT = E // NT
RB = 1000
G = N // RB

_MESH = plsc.VectorSubcoreMesh(
    core_axis_name="c", subcore_axis_name="s", num_cores=2, num_subcores=16)


def _mm(a, b):
    return jnp.dot(a, b, preferred_element_type=jnp.float32,
                   precision=lax.Precision.HIGHEST)


def _deg_body(lh_dst, rh_dst, ones_hbm, deg_lh, deg_rh, didx, onev, acc):
    c = lax.axis_index("c")
    s = lax.axis_index("s")
    CD = 2000

    def run(dst_hbm, deg_hbm):
        pltpu.sync_copy(ones_hbm, onev)

        @pl.when(s < 10)
        def _():
            pltpu.sync_copy(onev.at[pl.ds(0, 1000)],
                            acc.at[pl.ds(s * 1000, 1000)])
        plsc.subcore_barrier()

        @pl.loop(0, EPT // CD)
        def _(g):
            base = s * EPT + g * CD
            pltpu.sync_copy(dst_hbm.at[pl.ds(base, CD)], didx)
            pltpu.sync_copy(onev, acc.at[didx], add=True)

        plsc.subcore_barrier()

        @pl.when(s < 10)
        def _():
            pltpu.sync_copy(acc.at[pl.ds(s * 1000, 1000)],
                            onev.at[pl.ds(0, 1000)])
            pltpu.sync_copy(onev.at[pl.ds(0, 1000)],
                            deg_hbm.at[pl.ds(s * 1000, 1000)])

    pl.when(c == 0)(lambda: run(lh_dst, deg_lh))
    pl.when(c == 1)(lambda: run(rh_dst, deg_rh))


@functools.cache
def _deg_kernel():
    return pl.kernel(
        _deg_body,
        out_type=[jax.ShapeDtypeStruct((N,), jnp.float32),
                  jax.ShapeDtypeStruct((N,), jnp.float32)],
        mesh=_MESH,
        scratch_types=[
            pltpu.VMEM((2000,), jnp.int32),
            pltpu.VMEM((2000,), jnp.float32),
            pltpu.VMEM_SHARED((N,), jnp.float32),
        ],
    )


_SEG_C = 160
HN = N // 2


def _segsum_body(hs, src, dst, zrow, out, sidx0, didx0, didx0b,
                 sidx1, didx1, didx1b, rows0, rows1, zbuf, acc, sem0, semg1,
                 sem1):
    c = lax.axis_index("c")
    s = lax.axis_index("s")
    C = _SEG_C
    base_node = c * HN
    dump = HN + lax.iota(jnp.int32, 16)

    pltpu.sync_copy(zrow, zbuf)
    for j in range(HN // 200):
        @pl.when(s == j % NT)
        def _():
            pltpu.sync_copy(zbuf, acc.at[pl.ds(j * 200, 200)])

    iters = EPT // C
    plsc.subcore_barrier()

    def start_idx(g, sidx, didx, sem):
        base = s * EPT + g * C
        da = pltpu.async_copy(src.at[pl.ds(base, C)], sidx, sem)
        db = pltpu.async_copy(dst.at[pl.ds(base, C)], didx, sem)
        return da, db

    def localize(didx, didxb):
        @pl.loop(0, C // 16)
        def _(i):
            d = didx[pl.ds(i * 16, 16)] - base_node
            ok = (d >= 0) & (d < HN)
            didxb[pl.ds(i * 16, 16)] = jnp.where(ok, d, dump)

    da, db = start_idx(0, sidx0, didx0, sem1)
    da.wait()
    db.wait()
    localize(didx0, didx0b)

    @pl.loop(0, iters, step=2)
    def _(g):
        d0 = pltpu.async_copy(hs.at[sidx0], rows0, sem0)

        @pl.when(g + 1 < iters)
        def _():
            da, db = start_idx(g + 1, sidx1, didx1, sem1)
            da.wait()
            db.wait()
            localize(didx1, didx1b)
        d0.wait()

        @pl.when(g + 1 < iters)
        def _():
            d1 = pltpu.async_copy(hs.at[sidx1], rows1, semg1)
            pltpu.sync_copy(rows0, acc.at[didx0b], add=True)

            @pl.when(g + 2 < iters)
            def _():
                da, db = start_idx(g + 2, sidx0, didx0, sem1)
                da.wait()
                db.wait()
                localize(didx0, didx0b)
            d1.wait()
            pltpu.sync_copy(rows1, acc.at[didx1b], add=True)

        @pl.when(g + 1 >= iters)
        def _():
            pltpu.sync_copy(rows0, acc.at[didx0b], add=True)

    plsc.subcore_barrier()
    for j in range(HN // 200):
        @pl.when(s == j % NT)
        def _():
            pltpu.sync_copy(acc.at[pl.ds(j * 200, 200)], zbuf)
            pltpu.sync_copy(zbuf, out.at[pl.ds(c * HN + j * 200, 200)])


@functools.cache
def _segsum_kernel():
    C = _SEG_C
    return pl.kernel(
        _segsum_body,
        out_type=jax.ShapeDtypeStruct((N, 128), jnp.float32),
        mesh=_MESH,
        scratch_types=[
            pltpu.VMEM((C,), jnp.int32),
            pltpu.VMEM((C,), jnp.int32),
            pltpu.VMEM((C,), jnp.int32),
            pltpu.VMEM((C,), jnp.int32),
            pltpu.VMEM((C,), jnp.int32),
            pltpu.VMEM((C,), jnp.int32),
            pltpu.VMEM((C, 128), jnp.float32),
            pltpu.VMEM((C, 128), jnp.float32),
            pltpu.VMEM((200, 128), jnp.float32),
            pltpu.VMEM_SHARED((HN + 16, 128), jnp.float32),
            pltpu.SemaphoreType.DMA,
            pltpu.SemaphoreType.DMA,
            pltpu.SemaphoreType.DMA,
        ],
    )


def _segsum(hs, src, dst):
    zrow = jnp.zeros((200, 128), jnp.float32)
    return _segsum_kernel()(hs, src, dst, zrow)


def _enc_body(x, deg, gW1, gb1, gW2, gb2, mW1, mb1, mW2, mb2, h0, dinv):
    xb = x[...]
    eg = _mm(jax.nn.relu(_mm(xb[:, :GEO], gW1[...]) + gb1[...]), gW2[...]) + gb2[...]
    em = _mm(jax.nn.relu(_mm(xb[:, GEO:], mW1[...]) + mb1[...]), mW2[...]) + mb2[...]
    h0[...] = eg + em
    dinv[0, 0, :] = lax.rsqrt(deg[0, 0, :])


def _enc_call(x, deg3, p):
    wspec = lambda a: pl.BlockSpec(a.shape, lambda i: (0,) * a.ndim)
    args = [p["geo_W1"], p["geo_b1"], p["geo_W2"], p["geo_b2"],
            p["mor_W1"], p["mor_b1"], p["mor_W2"], p["mor_b2"]]
    return pl.pallas_call(
        _enc_body,
        grid=(G,),
        in_specs=[pl.BlockSpec((RB, IN_DIM), lambda i: (i, 0)),
                  pl.BlockSpec((1, 1, RB), lambda i: (i, 0, 0))] +
                 [wspec(a) for a in args],
        out_specs=[pl.BlockSpec((RB, HID), lambda i: (i, 0)),
                   pl.BlockSpec((1, 1, RB), lambda i: (i, 0, 0))],
        out_shape=[jax.ShapeDtypeStruct((N, HID), jnp.float32),
                   jax.ShapeDtypeStruct((G, 1, RB), jnp.float32)],
    )(x, deg3, *args)


def _pre_body(gated, x, dinv, W, gate, hs_outs, xg):
    xb = x[...]
    if gated:
        xb = xb * gate[...]
        xg[...] = xb
    hs = _mm(xb, W[...]) * dinv[0, 0, :][:, None]
    co = hs.shape[1]
    if co <= 128:
        if co < 128:
            hs = jnp.concatenate(
                [hs, jnp.zeros((hs.shape[0], 128 - co), jnp.float32)], axis=1)
        hs_outs[0][...] = hs
    else:
        hs_outs[0][...] = hs[:, :128]
        hs_outs[1][...] = hs[:, 128:]


def _pre_call(x, dinv3, W, gate=None):
    ci, co = W.shape
    n_hs = 1 if co <= 128 else 2
    gated = gate is not None
    in_specs = [pl.BlockSpec((RB, ci), lambda i: (i, 0)),
                pl.BlockSpec((1, 1, RB), lambda i: (i, 0, 0)),
                pl.BlockSpec((ci, co), lambda i: (0, 0))]
    args = [x, dinv3, W]
    out_specs = [pl.BlockSpec((RB, 128), lambda i: (i, 0))] * n_hs
    out_shape = [jax.ShapeDtypeStruct((N, 128), jnp.float32)] * n_hs
    if gated:
        in_specs.append(pl.BlockSpec((1, ci), lambda i: (0, 0)))
        args.append(gate)
        out_specs = out_specs + [pl.BlockSpec((RB, ci), lambda i: (i, 0))]
        out_shape = out_shape + [jax.ShapeDtypeStruct((N, ci), jnp.float32)]

    def body(*refs):
        ins = refs[:3 + gated]
        outs = refs[3 + gated:]
        hs_outs = outs[:n_hs]
        xg = outs[n_hs] if gated else None
        _pre_body(gated, ins[0], ins[1], ins[2],
                  ins[3] if gated else None, hs_outs, xg)

    res = pl.pallas_call(
        body, grid=(G,), in_specs=in_specs, out_specs=out_specs,
        out_shape=out_shape)(*args)
    if gated:
        return list(res[:n_hs]), res[n_hs]
    return list(res), None


_BN_C = 1.0 / (1.0 + 1e-5) ** 0.5


def _post_body(n_half, co, has_skip, has_pp, *refs):
    halves = [refs[2 * k:2 * k + 2] for k in range(n_half)]
    rest = refs[2 * n_half:]
    dinv, x, Wr, b, g, be = rest[:6]
    rest = rest[6:]
    skip = rest[0] if has_skip else None
    rest = rest[1:] if has_skip else rest
    out = rest[0]
    pp = rest[1] if has_pp else None
    dv = dinv[0, 0, :][:, None]
    cols = []
    for k, (agg, hs) in enumerate(halves):
        w = min(128, co - 128 * k)
        cols.append((agg[...] + hs[...])[:, :w])
    t = (jnp.concatenate(cols, axis=1) if len(cols) > 1 else cols[0]) * dv
    y = jax.nn.relu((t + b[...]) * (g[...] * _BN_C) + be[...]) + _mm(x[...], Wr[...])
    if has_skip:
        y = y + skip[...]
    out[...] = y
    if has_pp:
        pp[0, 0, :] = jnp.sum(y, axis=0)


def _post_call(partials, hs_list, dinv3, x, Wr, b, g, be,
               skip=None, want_pp=False):
    ci, co = Wr.shape
    n_half = len(hs_list)
    full = pl.BlockSpec((RB, 128), lambda i: (i, 0))
    vec = pl.BlockSpec((1, co), lambda i: (0, 0))
    in_specs, args = [], []
    for agg, hs in zip(partials, hs_list):
        in_specs += [full, full]
        args += [agg, hs]
    in_specs += [pl.BlockSpec((1, 1, RB), lambda i: (i, 0, 0)),
                 pl.BlockSpec((RB, ci), lambda i: (i, 0)),
                 pl.BlockSpec((ci, co), lambda i: (0, 0)),
                 vec, vec, vec]
    args += [dinv3, x, Wr, b.reshape(1, co), g.reshape(1, co),
             be.reshape(1, co)]
    if skip is not None:
        in_specs.append(pl.BlockSpec((RB, co), lambda i: (i, 0)))
        args.append(skip)
    out_specs = [pl.BlockSpec((RB, co), lambda i: (i, 0))]
    out_shape = [jax.ShapeDtypeStruct((N, co), jnp.float32)]
    if want_pp:
        out_specs.append(pl.BlockSpec((1, 1, co), lambda i: (i, 0, 0)))
        out_shape.append(jax.ShapeDtypeStruct((G, 1, co), jnp.float32))
    res = pl.pallas_call(
        functools.partial(_post_body, n_half, co, skip is not None, want_pp),
        grid=(G,), in_specs=in_specs, out_specs=out_specs,
        out_shape=out_shape)(*args)
    return res if want_pp else res[0]


def _rec_body(x, W1, b1, W2, b2, out):
    out[...] = _mm(jax.nn.relu(_mm(x[...], W1[...]) + b1[...]), W2[...]) + b2[...]


def _rec_call(x, W1, b1, W2, b2):
    ci = W1.shape[0]
    hid = W1.shape[1]
    co = W2.shape[1]
    return pl.pallas_call(
        _rec_body,
        grid=(G,),
        in_specs=[pl.BlockSpec((RB, ci), lambda i: (i, 0)),
                  pl.BlockSpec((ci, hid), lambda i: (0, 0)),
                  pl.BlockSpec((1, hid), lambda i: (0, 0)),
                  pl.BlockSpec((hid, co), lambda i: (0, 0)),
                  pl.BlockSpec((1, co), lambda i: (0, 0))],
        out_specs=pl.BlockSpec((RB, co), lambda i: (i, 0)),
        out_shape=jax.ShapeDtypeStruct((N, co), jnp.float32),
    )(x, W1, b1.reshape(1, hid), W2, b2.reshape(1, co))


def _gate_body(pp, W1, b1, W2, b2, gate):
    pooled = jnp.sum(pp[...].reshape(G, -1), axis=0, keepdims=True) * (1.0 / N)
    gate[...] = jax.nn.sigmoid(
        _mm(jax.nn.relu(_mm(pooled, W1[...]) + b1[...]), W2[...]) + b2[...])


def _gate_call(pp, W1, b1, W2, b2):
    d = W1.shape[0]
    h = W1.shape[1]
    return pl.pallas_call(
        _gate_body,
        out_shape=jax.ShapeDtypeStruct((1, d), jnp.float32),
    )(pp, W1, b1.reshape(1, h), W2, b2.reshape(1, d))


def _head_body(ppl, ppr, W1, b1, W2, b2, zl, zr, zg):
    l = jnp.sum(ppl[...].reshape(G, -1), axis=0, keepdims=True) * (1.0 / N)
    r = jnp.sum(ppr[...].reshape(G, -1), axis=0, keepdims=True) * (1.0 / N)
    zl[...] = l
    zr[...] = r
    zc = jnp.concatenate([l, r], axis=1)
    zg[...] = _mm(jax.nn.relu(_mm(zc, W1[...]) + b1[...]), W2[...]) + b2[...]


def _head_call(ppl, ppr, W1, b1, W2, b2):
    return pl.pallas_call(
        _head_body,
        out_shape=[jax.ShapeDtypeStruct((1, HID), jnp.float32),
                   jax.ShapeDtypeStruct((1, HID), jnp.float32),
                   jax.ShapeDtypeStruct((1, 128), jnp.float32)],
    )(ppl, ppr, W1, b1.reshape(1, -1), W2, b2.reshape(1, -1))


def _gcn_block(p, pre, x, src, dst, dinv3, gate=None, skip=None, want_pp=False):
    hs_list, xg = _pre_call(x, dinv3, p[pre + "_W"], gate)
    if gate is not None:
        x = xg
    partials = [_segsum(hs, src, dst) for hs in hs_list]
    res = _post_call(partials, hs_list, dinv3, x, p[pre + "_Wr"],
                     p[pre + "_b"], p[pre + "_g"], p[pre + "_be"],
                     skip=skip, want_pp=want_pp)
    if gate is not None:
        return (res, x)
    return res


def _hemi(p, x, src, dst, deg):
    deg3 = deg.reshape(G, 1, RB)
    h0, dinv3 = _enc_call(x, deg3, p)
    h1, pp1 = _gcn_block(p, "b1", h0, src, dst, dinv3, want_pp=True)
    gate = _gate_call(pp1, p["se_W1"], p["se_b1"], p["se_W2"], p["se_b2"])
    (h2, h1g) = _gcn_block(p, "b2", h1, src, dst, dinv3, gate=gate)
    h3 = _gcn_block(p, "b3", h2, src, dst, dinv3)
    u2 = _gcn_block(p, "b4", h3, src, dst, dinv3, skip=h2)
    u1 = _gcn_block(p, "b5", u2, src, dst, dinv3, skip=h1g)
    u0, ppz = _gcn_block(p, "b6", u1, src, dst, dinv3, skip=h0, want_pp=True)
    recon = _rec_call(u0, p["rec_W1"], p["rec_b1"], p["rec_W2"], p["rec_b2"])
    return u0, recon, ppz


def kernel(params, lh_x, rh_x, lh_edge_index, rh_edge_index):
    p = params
    lh_src, lh_dst = lh_edge_index[0], lh_edge_index[1]
    rh_src, rh_dst = rh_edge_index[0], rh_edge_index[1]
    ones = jnp.ones((2000,), jnp.float32)
    deg_lh, deg_rh = _deg_kernel()(lh_dst, rh_dst, ones)
    lh_Hv, lh_recon, lh_ppz = _hemi(p, lh_x, lh_src, lh_dst, deg_lh)
    rh_Hv, rh_recon, rh_ppz = _hemi(p, rh_x, rh_src, rh_dst, deg_rh)
    zl, zr, zg = _head_call(lh_ppz, rh_ppz,
                            p["gh_W1"], p["gh_b1"], p["gh_W2"], p["gh_b2"])
    return {"lh_Hv": lh_Hv, "lh_recon": lh_recon, "lh_z": zl.reshape(HID),
            "rh_Hv": rh_Hv, "rh_recon": rh_recon, "rh_z": zr.reshape(HID),
            "z_graph": zg.reshape(128)}


# speedup vs baseline: 1.0025x; 1.0025x over previous
import functools

import jax
import jax.numpy as jnp
from jax import lax
from jax.experimental import pallas as pl
from jax.experimental.pallas import tpu as pltpu
from jax.experimental.pallas import tpu_sc as plsc

N = 10000
E = 320000
IN_DIM = 128
GEO = 64
HID = 32
DIMS = [32, 64, 128, 256, 128, 64, 32]

NT = 16
EPT = E // NT
RB = 1000
G = N // RB

_MESH = plsc.VectorSubcoreMesh(
    core_axis_name="c", subcore_axis_name="s", num_cores=2, num_subcores=16)


def _mm(a, b):
    return jnp.dot(a, b, preferred_element_type=jnp.float32,
                   precision=lax.Precision.HIGHEST)


def _deg_body(lh_dst, rh_dst, ones_hbm, deg_lh, deg_rh, didx, onev, acc):
    c = lax.axis_index("c")
    s = lax.axis_index("s")
    CD = 2000

    def run(dst_hbm, deg_hbm):
        pltpu.sync_copy(ones_hbm, onev)

        @pl.when(s < 10)
        def _():
            pltpu.sync_copy(onev.at[pl.ds(0, 1000)],
                            acc.at[pl.ds(s * 1000, 1000)])
        plsc.subcore_barrier()

        @pl.loop(0, EPT // CD)
        def _(g):
            base = s * EPT + g * CD
            pltpu.sync_copy(dst_hbm.at[pl.ds(base, CD)], didx)
            pltpu.sync_copy(onev, acc.at[didx], add=True)

        plsc.subcore_barrier()

        @pl.when(s < 10)
        def _():
            pltpu.sync_copy(acc.at[pl.ds(s * 1000, 1000)],
                            onev.at[pl.ds(0, 1000)])
            pltpu.sync_copy(onev.at[pl.ds(0, 1000)],
                            deg_hbm.at[pl.ds(s * 1000, 1000)])

    pl.when(c == 0)(lambda: run(lh_dst, deg_lh))
    pl.when(c == 1)(lambda: run(rh_dst, deg_rh))


@functools.cache
def _deg_kernel():
    return pl.kernel(
        _deg_body,
        out_type=[jax.ShapeDtypeStruct((N,), jnp.float32),
                  jax.ShapeDtypeStruct((N,), jnp.float32)],
        mesh=_MESH,
        scratch_types=[
            pltpu.VMEM((2000,), jnp.int32),
            pltpu.VMEM((2000,), jnp.float32),
            pltpu.VMEM_SHARED((N,), jnp.float32),
        ],
    )


_SEG_C = 400
HN = N // 2


def _segsum_body(hs, src, dst, zrow, out, sidx0, didx0, didx0b,
                 sidx1, didx1, didx1b, rows0, zbuf, acc, sem0, sem1):
    c = lax.axis_index("c")
    s = lax.axis_index("s")
    C = _SEG_C
    base_node = c * HN
    dump = HN + lax.iota(jnp.int32, 16)

    pltpu.sync_copy(zrow, zbuf)
    for j in range(HN // 200):
        @pl.when(s == j % NT)
        def _():
            pltpu.sync_copy(zbuf, acc.at[pl.ds(j * 200, 200)])

    iters = EPT // C
    plsc.subcore_barrier()

    def start_idx(g, sidx, didx, sem):
        base = s * EPT + g * C
        da = pltpu.async_copy(src.at[pl.ds(base, C)], sidx, sem)
        db = pltpu.async_copy(dst.at[pl.ds(base, C)], didx, sem)
        return da, db

    def localize(didx, didxb):
        @pl.loop(0, C // 16)
        def _(i):
            d = didx[pl.ds(i * 16, 16)] - base_node
            ok = (d >= 0) & (d < HN)
            didxb[pl.ds(i * 16, 16)] = jnp.where(ok, d, dump)

    def step(sidx_c, didxb_c, nidx):
        d = pltpu.async_copy(hs.at[sidx_c], rows0, sem0)
        if nidx is not None:
            g1, sidx_n, didx_n, didxb_n, semi = nidx

            @pl.when(g1 < iters)
            def _():
                da, db = start_idx(g1, sidx_n, didx_n, semi)
                da.wait()
                db.wait()
                localize(didx_n, didxb_n)
        d.wait()
        pltpu.sync_copy(rows0, acc.at[didxb_c], add=True)

    da, db = start_idx(0, sidx0, didx0, sem1)
    da.wait()
    db.wait()
    localize(didx0, didx0b)

    @pl.loop(0, iters, step=2)
    def _(g):
        step(sidx0, didx0b, (g + 1, sidx1, didx1, didx1b, sem1))
        step(sidx1, didx1b, (g + 2, sidx0, didx0, didx0b, sem1))

    plsc.subcore_barrier()
    for j in range(HN // 200):
        @pl.when(s == j % NT)
        def _():
            pltpu.sync_copy(acc.at[pl.ds(j * 200, 200)], zbuf)
            pltpu.sync_copy(zbuf, out.at[pl.ds(c * HN + j * 200, 200)])


@functools.cache
def _segsum_kernel():
    C = _SEG_C
    return pl.kernel(
        _segsum_body,
        out_type=jax.ShapeDtypeStruct((N, 128), jnp.float32),
        mesh=_MESH,
        scratch_types=[
            pltpu.VMEM((C,), jnp.int32),
            pltpu.VMEM((C,), jnp.int32),
            pltpu.VMEM((C,), jnp.int32),
            pltpu.VMEM((C,), jnp.int32),
            pltpu.VMEM((C,), jnp.int32),
            pltpu.VMEM((C,), jnp.int32),
            pltpu.VMEM((C, 128), jnp.float32),
            pltpu.VMEM((200, 128), jnp.float32),
            pltpu.VMEM_SHARED((HN + 16, 128), jnp.float32),
            pltpu.SemaphoreType.DMA,
            pltpu.SemaphoreType.DMA,
        ],
    )


def _segsum(hs, src, dst):
    zrow = jnp.zeros((200, 128), jnp.float32)
    return _segsum_kernel()(hs, src, dst, zrow)


def _enc_body(x, deg, gW1, gb1, gW2, gb2, mW1, mb1, mW2, mb2, h0, dinv):
    xb = x[...]
    eg = _mm(jax.nn.relu(_mm(xb[:, :GEO], gW1[...]) + gb1[...]), gW2[...]) + gb2[...]
    em = _mm(jax.nn.relu(_mm(xb[:, GEO:], mW1[...]) + mb1[...]), mW2[...]) + mb2[...]
    h0[...] = eg + em
    dinv[0, 0, :] = lax.rsqrt(deg[0, 0, :])


def _enc_call(x, deg3, p):
    wspec = lambda a: pl.BlockSpec(a.shape, lambda i: (0,) * a.ndim)
    args = [p["geo_W1"], p["geo_b1"], p["geo_W2"], p["geo_b2"],
            p["mor_W1"], p["mor_b1"], p["mor_W2"], p["mor_b2"]]
    return pl.pallas_call(
        _enc_body,
        grid=(G,),
        in_specs=[pl.BlockSpec((RB, IN_DIM), lambda i: (i, 0)),
                  pl.BlockSpec((1, 1, RB), lambda i: (i, 0, 0))] +
                 [wspec(a) for a in args],
        out_specs=[pl.BlockSpec((RB, HID), lambda i: (i, 0)),
                   pl.BlockSpec((1, 1, RB), lambda i: (i, 0, 0))],
        out_shape=[jax.ShapeDtypeStruct((N, HID), jnp.float32),
                   jax.ShapeDtypeStruct((G, 1, RB), jnp.float32)],
    )(x, deg3, *args)


def _pre_body(gated, x, dinv, W, gate, hs_outs, xg):
    xb = x[...]
    if gated:
        xb = xb * gate[...]
        xg[...] = xb
    hs = _mm(xb, W[...]) * dinv[0, 0, :][:, None]
    co = hs.shape[1]
    if co <= 128:
        if co < 128:
            hs = jnp.concatenate(
                [hs, jnp.zeros((hs.shape[0], 128 - co), jnp.float32)], axis=1)
        hs_outs[0][...] = hs
    else:
        hs_outs[0][...] = hs[:, :128]
        hs_outs[1][...] = hs[:, 128:]


def _pre_call(x, dinv3, W, gate=None):
    ci, co = W.shape
    n_hs = 1 if co <= 128 else 2
    gated = gate is not None
    in_specs = [pl.BlockSpec((RB, ci), lambda i: (i, 0)),
                pl.BlockSpec((1, 1, RB), lambda i: (i, 0, 0)),
                pl.BlockSpec((ci, co), lambda i: (0, 0))]
    args = [x, dinv3, W]
    out_specs = [pl.BlockSpec((RB, 128), lambda i: (i, 0))] * n_hs
    out_shape = [jax.ShapeDtypeStruct((N, 128), jnp.float32)] * n_hs
    if gated:
        in_specs.append(pl.BlockSpec((1, ci), lambda i: (0, 0)))
        args.append(gate)
        out_specs = out_specs + [pl.BlockSpec((RB, ci), lambda i: (i, 0))]
        out_shape = out_shape + [jax.ShapeDtypeStruct((N, ci), jnp.float32)]

    def body(*refs):
        ins = refs[:3 + gated]
        outs = refs[3 + gated:]
        hs_outs = outs[:n_hs]
        xg = outs[n_hs] if gated else None
        _pre_body(gated, ins[0], ins[1], ins[2],
                  ins[3] if gated else None, hs_outs, xg)

    res = pl.pallas_call(
        body, grid=(G,), in_specs=in_specs, out_specs=out_specs,
        out_shape=out_shape)(*args)
    if gated:
        return list(res[:n_hs]), res[n_hs]
    return list(res), None


_BN_C = 1.0 / (1.0 + 1e-5) ** 0.5


def _post_body(n_half, co, has_skip, has_pp, *refs):
    halves = [refs[2 * k:2 * k + 2] for k in range(n_half)]
    rest = refs[2 * n_half:]
    dinv, x, Wr, b, g, be = rest[:6]
    rest = rest[6:]
    skip = rest[0] if has_skip else None
    rest = rest[1:] if has_skip else rest
    out = rest[0]
    pp = rest[1] if has_pp else None
    dv = dinv[0, 0, :][:, None]
    cols = []
    for k, (agg, hs) in enumerate(halves):
        w = min(128, co - 128 * k)
        cols.append((agg[...] + hs[...])[:, :w])
    t = (jnp.concatenate(cols, axis=1) if len(cols) > 1 else cols[0]) * dv
    y = jax.nn.relu((t + b[...]) * (g[...] * _BN_C) + be[...]) + _mm(x[...], Wr[...])
    if has_skip:
        y = y + skip[...]
    out[...] = y
    if has_pp:
        pp[0, 0, :] = jnp.sum(y, axis=0)


def _post_call(partials, hs_list, dinv3, x, Wr, b, g, be,
               skip=None, want_pp=False):
    ci, co = Wr.shape
    n_half = len(hs_list)
    full = pl.BlockSpec((RB, 128), lambda i: (i, 0))
    vec = pl.BlockSpec((1, co), lambda i: (0, 0))
    in_specs, args = [], []
    for agg, hs in zip(partials, hs_list):
        in_specs += [full, full]
        args += [agg, hs]
    in_specs += [pl.BlockSpec((1, 1, RB), lambda i: (i, 0, 0)),
                 pl.BlockSpec((RB, ci), lambda i: (i, 0)),
                 pl.BlockSpec((ci, co), lambda i: (0, 0)),
                 vec, vec, vec]
    args += [dinv3, x, Wr, b.reshape(1, co), g.reshape(1, co),
             be.reshape(1, co)]
    if skip is not None:
        in_specs.append(pl.BlockSpec((RB, co), lambda i: (i, 0)))
        args.append(skip)
    out_specs = [pl.BlockSpec((RB, co), lambda i: (i, 0))]
    out_shape = [jax.ShapeDtypeStruct((N, co), jnp.float32)]
    if want_pp:
        out_specs.append(pl.BlockSpec((1, 1, co), lambda i: (i, 0, 0)))
        out_shape.append(jax.ShapeDtypeStruct((G, 1, co), jnp.float32))
    res = pl.pallas_call(
        functools.partial(_post_body, n_half, co, skip is not None, want_pp),
        grid=(G,), in_specs=in_specs, out_specs=out_specs,
        out_shape=out_shape)(*args)
    return res if want_pp else res[0]


def _rec_body(x, W1, b1, W2, b2, out):
    out[...] = _mm(jax.nn.relu(_mm(x[...], W1[...]) + b1[...]), W2[...]) + b2[...]


def _rec_call(x, W1, b1, W2, b2):
    ci = W1.shape[0]
    hid = W1.shape[1]
    co = W2.shape[1]
    return pl.pallas_call(
        _rec_body,
        grid=(G,),
        in_specs=[pl.BlockSpec((RB, ci), lambda i: (i, 0)),
                  pl.BlockSpec((ci, hid), lambda i: (0, 0)),
                  pl.BlockSpec((1, hid), lambda i: (0, 0)),
                  pl.BlockSpec((hid, co), lambda i: (0, 0)),
                  pl.BlockSpec((1, co), lambda i: (0, 0))],
        out_specs=pl.BlockSpec((RB, co), lambda i: (i, 0)),
        out_shape=jax.ShapeDtypeStruct((N, co), jnp.float32),
    )(x, W1, b1.reshape(1, hid), W2, b2.reshape(1, co))


def _gate_body(pp, W1, b1, W2, b2, gate):
    pooled = jnp.sum(pp[...].reshape(G, -1), axis=0, keepdims=True) * (1.0 / N)
    gate[...] = jax.nn.sigmoid(
        _mm(jax.nn.relu(_mm(pooled, W1[...]) + b1[...]), W2[...]) + b2[...])


def _gate_call(pp, W1, b1, W2, b2):
    d = W1.shape[0]
    h = W1.shape[1]
    return pl.pallas_call(
        _gate_body,
        out_shape=jax.ShapeDtypeStruct((1, d), jnp.float32),
    )(pp, W1, b1.reshape(1, h), W2, b2.reshape(1, d))


def _head_body(ppl, ppr, W1, b1, W2, b2, zl, zr, zg):
    l = jnp.sum(ppl[...].reshape(G, -1), axis=0, keepdims=True) * (1.0 / N)
    r = jnp.sum(ppr[...].reshape(G, -1), axis=0, keepdims=True) * (1.0 / N)
    zl[...] = l
    zr[...] = r
    zc = jnp.concatenate([l, r], axis=1)
    zg[...] = _mm(jax.nn.relu(_mm(zc, W1[...]) + b1[...]), W2[...]) + b2[...]


def _head_call(ppl, ppr, W1, b1, W2, b2):
    return pl.pallas_call(
        _head_body,
        out_shape=[jax.ShapeDtypeStruct((1, HID), jnp.float32),
                   jax.ShapeDtypeStruct((1, HID), jnp.float32),
                   jax.ShapeDtypeStruct((1, 128), jnp.float32)],
    )(ppl, ppr, W1, b1.reshape(1, -1), W2, b2.reshape(1, -1))


def _gcn_block(p, pre, x, src, dst, dinv3, gate=None, skip=None, want_pp=False):
    hs_list, xg = _pre_call(x, dinv3, p[pre + "_W"], gate)
    if gate is not None:
        x = xg
    partials = [_segsum(hs, src, dst) for hs in hs_list]
    res = _post_call(partials, hs_list, dinv3, x, p[pre + "_Wr"],
                     p[pre + "_b"], p[pre + "_g"], p[pre + "_be"],
                     skip=skip, want_pp=want_pp)
    if gate is not None:
        return (res, x)
    return res


def _hemi(p, x, src, dst, deg):
    deg3 = deg.reshape(G, 1, RB)
    h0, dinv3 = _enc_call(x, deg3, p)
    h1, pp1 = _gcn_block(p, "b1", h0, src, dst, dinv3, want_pp=True)
    gate = _gate_call(pp1, p["se_W1"], p["se_b1"], p["se_W2"], p["se_b2"])
    (h2, h1g) = _gcn_block(p, "b2", h1, src, dst, dinv3, gate=gate)
    h3 = _gcn_block(p, "b3", h2, src, dst, dinv3)
    u2 = _gcn_block(p, "b4", h3, src, dst, dinv3, skip=h2)
    u1 = _gcn_block(p, "b5", u2, src, dst, dinv3, skip=h1g)
    u0, ppz = _gcn_block(p, "b6", u1, src, dst, dinv3, skip=h0, want_pp=True)
    recon = _rec_call(u0, p["rec_W1"], p["rec_b1"], p["rec_W2"], p["rec_b2"])
    return u0, recon, ppz


def kernel(params, lh_x, rh_x, lh_edge_index, rh_edge_index):
    p = params
    lh_src, lh_dst = lh_edge_index[0], lh_edge_index[1]
    rh_src, rh_dst = rh_edge_index[0], rh_edge_index[1]
    ones = jnp.ones((2000,), jnp.float32)
    deg_lh, deg_rh = _deg_kernel()(lh_dst, rh_dst, ones)
    lh_Hv, lh_recon, lh_ppz = _hemi(p, lh_x, lh_src, lh_dst, deg_lh)
    rh_Hv, rh_recon, rh_ppz = _hemi(p, rh_x, rh_src, rh_dst, deg_rh)
    zl, zr, zg = _head_call(lh_ppz, rh_ppz,
                            p["gh_W1"], p["gh_b1"], p["gh_W2"], p["gh_b2"])
    return {"lh_Hv": lh_Hv, "lh_recon": lh_recon, "lh_z": zl.reshape(HID),
            "rh_Hv": rh_Hv, "rh_recon": rh_recon, "rh_z": zr.reshape(HID),
            "z_graph": zg.reshape(128)}
